# baseline (speedup 1.0000x reference)
import functools

import jax
import jax.numpy as jnp
from jax import lax
from jax.experimental import pallas as pl
from jax.experimental.pallas import tpu as pltpu
from jax.experimental.pallas import tpu_sc as plsc

B, N, D = 16, 4096, 256
L = 16
NC, NS = 2, 16
DH = D // 2
G = DH // L
CH = 256
NCHUNK = N // CH

_mesh = plsc.VectorSubcoreMesh(core_axis_name="c", subcore_axis_name="s")


def _sqrt16(v):
    i = lax.bitcast_convert_type(v, jnp.int32)
    i = jnp.int32(0x5F3759DF) - (i >> 1)
    y = lax.bitcast_convert_type(i, jnp.float32)
    half_v = v * jnp.float32(0.5)
    for _ in range(3):
        y = y * (jnp.float32(1.5) - half_v * y * y)
    s = v * y
    return jnp.where(v > jnp.float32(0.0), s, jnp.float32(0.0))


@functools.partial(
    pl.kernel,
    mesh=_mesh,
    out_type=jax.ShapeDtypeStruct((B, 4, D), jnp.float32),
    scratch_types=[
        pltpu.VMEM((2, CH, DH), jnp.float32),
        pltpu.VMEM((4, DH), jnp.float32),
        pltpu.SemaphoreType.DMA,
        pltpu.SemaphoreType.DMA,
    ],
)
def _pool(x_hbm, out_hbm, buf, res, sem0, sem1):
    wid = lax.axis_index("s") * NC + lax.axis_index("c")
    b = wid // 2
    h = (wid % 2) * DH
    sems = (sem0, sem1)

    def _src(c):
        return x_hbm.at[b, pl.ds(c * CH, CH), pl.ds(h, DH)]

    pltpu.async_copy(_src(0), buf.at[0], sems[0])

    accs = []
    for _ in range(G):
        accs += [
            jnp.full((L,), -jnp.inf, jnp.float32),
            jnp.full((L,), jnp.inf, jnp.float32),
            jnp.zeros((L,), jnp.float32),
            jnp.zeros((L,), jnp.float32),
        ]
    accs = tuple(accs)

    for c in range(NCHUNK):
        slot = c % 2
        if c + 1 < NCHUNK:
            pltpu.async_copy(_src(c + 1), buf.at[1 - slot], sems[1 - slot])
        pltpu.make_async_copy(_src(c), buf.at[slot], sems[slot]).wait()
        vbuf = buf.at[slot]

        def body(r, a, vbuf=vbuf):
            out = []
            for g in range(G):
                v = vbuf[r, pl.ds(g * L, L)]
                mx, mn, sm, sq = a[4 * g : 4 * g + 4]
                out += [jnp.maximum(mx, v), jnp.minimum(mn, v),
                        sm + v, sq + v * v]
            return tuple(out)

        accs = lax.fori_loop(0, CH, body, accs)

    inv_n = jnp.float32(1.0 / N)
    inv_nm1 = jnp.float32(1.0 / (N - 1))
    for g in range(G):
        mx, mn, sm, sq = accs[4 * g : 4 * g + 4]
        mean = sm * inv_n
        var = (sq - sm * mean) * inv_nm1
        std = _sqrt16(jnp.maximum(var, jnp.float32(0.0)))
        sl = pl.ds(g * L, L)
        res[0, sl] = mx
        res[1, sl] = mn
        res[2, sl] = mean
        res[3, sl] = std

    pltpu.sync_copy(res, out_hbm.at[b, :, pl.ds(h, DH)])


def kernel(x):
    return _pool(x)


# speedup vs baseline: 1.0021x; 1.0021x over previous
import functools

import jax
import jax.numpy as jnp
from jax import lax
from jax.experimental import pallas as pl
from jax.experimental.pallas import tpu as pltpu
from jax.experimental.pallas import tpu_sc as plsc

B, N, D = 16, 4096, 256
L = 16
NC, NS = 2, 16
DH = D // 2
G = DH // L
CH = 256
NCHUNK = N // CH

_mesh = plsc.VectorSubcoreMesh(core_axis_name="c", subcore_axis_name="s")


def _sqrt16(v):
    i = lax.bitcast_convert_type(v, jnp.int32)
    i = jnp.int32(0x5F3759DF) - (i >> 1)
    y = lax.bitcast_convert_type(i, jnp.float32)
    half_v = v * jnp.float32(0.5)
    for _ in range(3):
        y = y * (jnp.float32(1.5) - half_v * y * y)
    s = v * y
    return jnp.where(v > jnp.float32(0.0), s, jnp.float32(0.0))


@functools.partial(
    pl.kernel,
    mesh=_mesh,
    out_type=jax.ShapeDtypeStruct((B, 4, D), jnp.float32),
    scratch_types=[
        pltpu.VMEM((2, CH, DH), jnp.float32),
        pltpu.VMEM((4, DH), jnp.float32),
        pltpu.SemaphoreType.DMA,
        pltpu.SemaphoreType.DMA,
    ],
)
def _pool(x_hbm, out_hbm, buf, res, sem0, sem1):
    wid = lax.axis_index("s") * NC + lax.axis_index("c")
    b = wid // 2
    h = (wid % 2) * DH
    sems = (sem0, sem1)

    def _src(c):
        return x_hbm.at[b, pl.ds(c * CH, CH), pl.ds(h, DH)]

    pltpu.async_copy(_src(0), buf.at[0], sems[0])

    accs = []
    for _ in range(G):
        accs += [
            jnp.full((L,), -jnp.inf, jnp.float32),
            jnp.full((L,), jnp.inf, jnp.float32),
            jnp.zeros((L,), jnp.float32),
            jnp.zeros((L,), jnp.float32),
        ]
    accs = tuple(accs)

    for c in range(NCHUNK):
        slot = c % 2
        if c + 1 < NCHUNK:
            pltpu.async_copy(_src(c + 1), buf.at[1 - slot], sems[1 - slot])
        pltpu.make_async_copy(_src(c), buf.at[slot], sems[slot]).wait()
        vbuf = buf.at[slot]

        def body(r, a, vbuf=vbuf):
            out = []
            for g in range(G):
                v = vbuf[r, pl.ds(g * L, L)]
                mx, mn, sm, sq = a[4 * g : 4 * g + 4]
                out += [jnp.maximum(mx, v), jnp.minimum(mn, v),
                        sm + v, sq + v * v]
            return tuple(out)

        accs = plsc.parallel_loop(0, CH, carry=accs, unroll=8)(body)

    inv_n = jnp.float32(1.0 / N)
    inv_nm1 = jnp.float32(1.0 / (N - 1))
    for g in range(G):
        mx, mn, sm, sq = accs[4 * g : 4 * g + 4]
        mean = sm * inv_n
        var = (sq - sm * mean) * inv_nm1
        std = _sqrt16(jnp.maximum(var, jnp.float32(0.0)))
        sl = pl.ds(g * L, L)
        res[0, sl] = mx
        res[1, sl] = mn
        res[2, sl] = mean
        res[3, sl] = std

    pltpu.sync_copy(res, out_hbm.at[b, :, pl.ds(h, DH)])


def kernel(x):
    return _pool(x)
